# Initial kernel scaffold; baseline (speedup 1.0000x reference)
import functools

import jax
import jax.numpy as jnp
from jax import lax
from jax.experimental import pallas as pl
from jax.experimental.pallas import tpu as pltpu
from jax.experimental.pallas import tpu_sc as plsc

N = 10000
E = 320000
D = 128
G = 128
L = 4
OUT = 128

NC = 2
NS = 16

CHUNK = 128
N_PAD = 10016
ROWS_PER_TILE = N_PAD // NS
EDGES_PER_TILE = 10112
E_PAD = EDGES_PER_TILE * NC * NS
CHUNKS_PER_TILE = EDGES_PER_TILE // CHUNK

_MESH = plsc.VectorSubcoreMesh(core_axis_name="c", subcore_axis_name="s")


def _deg_body(col_hbm, zeros_hbm, ones_hbm, out_hbm, shared, idx_v, ones_v, sem):
    c = lax.axis_index("c")
    s = lax.axis_index("s")
    wid = s * NC + c
    r0 = s * ROWS_PER_TILE
    pltpu.sync_copy(zeros_hbm.at[pl.ds(r0, ROWS_PER_TILE), :],
                    shared.at[pl.ds(r0, ROWS_PER_TILE), :])
    pltpu.sync_copy(ones_hbm, ones_v)
    plsc.subcore_barrier()
    base = wid * EDGES_PER_TILE

    def body(j, carry):
        pltpu.sync_copy(col_hbm.at[pl.ds(base + j * CHUNK, CHUNK)], idx_v)
        pltpu.sync_copy(ones_v, shared.at[idx_v], add=True)
        return carry

    lax.fori_loop(0, CHUNKS_PER_TILE, body, 0)
    plsc.subcore_barrier()
    pltpu.sync_copy(shared.at[pl.ds(r0, ROWS_PER_TILE), :],
                    out_hbm.at[pl.ds(c * N_PAD + r0, ROWS_PER_TILE), :])


_deg_kernel = functools.partial(
    pl.kernel,
    out_type=jax.ShapeDtypeStruct((NC * N_PAD, 8), jnp.float32),
    mesh=_MESH,
    scratch_types=[
        pltpu.VMEM_SHARED((N_PAD, 8), jnp.float32),
        pltpu.VMEM((CHUNK,), jnp.int32),
        pltpu.VMEM((CHUNK, 8), jnp.float32),
        pltpu.SemaphoreType.DMA,
    ],
)(_deg_body)


def _agg_body(hs_hbm, row_hbm, col_hbm, zeros_hbm, out_hbm,
              shared, ridx_v, cidx_v, rows_v, sem):
    c = lax.axis_index("c")
    s = lax.axis_index("s")
    wid = s * NC + c
    r0 = s * ROWS_PER_TILE
    pltpu.sync_copy(zeros_hbm.at[pl.ds(r0, ROWS_PER_TILE), :],
                    shared.at[pl.ds(r0, ROWS_PER_TILE), :])
    plsc.subcore_barrier()
    base = wid * EDGES_PER_TILE

    def body(j, carry):
        e0 = base + j * CHUNK
        pltpu.sync_copy(row_hbm.at[pl.ds(e0, CHUNK)], ridx_v)
        pltpu.sync_copy(col_hbm.at[pl.ds(e0, CHUNK)], cidx_v)
        pltpu.async_copy(hs_hbm.at[ridx_v], rows_v, sem).wait()
        pltpu.sync_copy(rows_v, shared.at[cidx_v], add=True)
        return carry

    lax.fori_loop(0, CHUNKS_PER_TILE, body, 0)
    plsc.subcore_barrier()
    pltpu.sync_copy(shared.at[pl.ds(r0, ROWS_PER_TILE), :],
                    out_hbm.at[pl.ds(c * N_PAD + r0, ROWS_PER_TILE), :])


_agg_kernel = functools.partial(
    pl.kernel,
    out_type=jax.ShapeDtypeStruct((NC * N_PAD, D), jnp.float32),
    mesh=_MESH,
    scratch_types=[
        pltpu.VMEM_SHARED((N_PAD, D), jnp.float32),
        pltpu.VMEM((CHUNK,), jnp.int32),
        pltpu.VMEM((CHUNK,), jnp.int32),
        pltpu.VMEM((CHUNK, D), jnp.float32),
        pltpu.SemaphoreType.DMA,
    ],
)(_agg_body)


def _proj_body(x_ref, wp_ref, bp_ref, degp_ref, h_ref, hs_ref, dis_ref):
    deg = degp_ref[0:N_PAD, 0:1] + degp_ref[N_PAD:2 * N_PAD, 0:1]
    dis = lax.rsqrt(jnp.maximum(deg, 1.0))
    dis_ref[...] = dis
    h = jnp.dot(x_ref[...], wp_ref[...], preferred_element_type=jnp.float32)
    h = h + bp_ref[...]
    h_ref[...] = h
    hs_ref[...] = dis[:N] * h


_proj_kernel = pl.pallas_call(
    _proj_body,
    out_shape=(
        jax.ShapeDtypeStruct((N, D), jnp.float32),
        jax.ShapeDtypeStruct((N, D), jnp.float32),
        jax.ShapeDtypeStruct((N_PAD, 1), jnp.float32),
    ),
)


def _layer_body(aggp_ref, dis_ref, h_ref, w_ref, b_ref, hn_ref, hsn_ref):
    agg = aggp_ref[0:N, :] + aggp_ref[N_PAD:N_PAD + N, :]
    agg = agg * dis_ref[0:N, :]
    out = jnp.dot(agg, w_ref[...], preferred_element_type=jnp.float32)
    out = jnp.maximum(out + b_ref[...], 0.0) + h_ref[...]
    hn_ref[...] = out
    hsn_ref[...] = dis_ref[0:N, :] * out


_layer_kernel = pl.pallas_call(
    _layer_body,
    out_shape=(
        jax.ShapeDtypeStruct((N, D), jnp.float32),
        jax.ShapeDtypeStruct((N, D), jnp.float32),
    ),
)


def _pool_body(h_ref, batch_ref, wr1_ref, br1_ref, wr2_ref, br2_ref, out_ref):
    gids = lax.broadcasted_iota(jnp.int32, (G, N), 0)
    onehot_t = jnp.where(gids == batch_ref[...], 1.0, 0.0)
    sums = jnp.dot(onehot_t, h_ref[...], preferred_element_type=jnp.float32)
    counts = jnp.dot(onehot_t, jnp.ones((N, 1), jnp.float32),
                     preferred_element_type=jnp.float32)
    emb = sums / jnp.maximum(counts, 1.0)
    hid = jnp.dot(emb, wr1_ref[...], preferred_element_type=jnp.float32)
    hid = jnp.maximum(hid + br1_ref[...], 0.0)
    out = jnp.dot(hid, wr2_ref[...], preferred_element_type=jnp.float32)
    out_ref[...] = out + br2_ref[...]


_pool_kernel = pl.pallas_call(
    _pool_body,
    out_shape=jax.ShapeDtypeStruct((G, OUT), jnp.float32),
)


def kernel(x, edge_index, edge_attr, batch, Wp, bp, Wls, bls, Wr1, br1, Wr2, br2):
    del edge_attr
    row = edge_index[0]
    col = edge_index[1]
    pad = E_PAD - E
    row_p = jnp.concatenate([row, jnp.zeros((pad,), jnp.int32)])
    col_p = jnp.concatenate([col, jnp.full((pad,), N, jnp.int32)])

    zeros_n8 = jnp.zeros((N_PAD, 8), jnp.float32)
    zeros_nd = jnp.zeros((N_PAD, D), jnp.float32)
    ones_k8 = jnp.ones((CHUNK, 8), jnp.float32)

    degp = _deg_kernel(col_p, zeros_n8, ones_k8)
    h, hs, dis = _proj_kernel(x, Wp, bp[None, :], degp)
    for i in range(L):
        aggp = _agg_kernel(hs, row_p, col_p, zeros_nd)
        h, hs = _layer_kernel(aggp, dis, h, Wls[i], bls[i][None, :])
    return _pool_kernel(h, batch[None, :], Wr1, br1[None, :], Wr2, br2[None, :])


# speedup vs baseline: 6.1980x; 6.1980x over previous
import functools

import jax
import jax.numpy as jnp
from jax import lax
from jax.experimental import pallas as pl
from jax.experimental.pallas import tpu as pltpu
from jax.experimental.pallas import tpu_sc as plsc

N = 10000
E = 320000
D = 128
G = 128
L = 4
OUT = 128

NC = 2
NS = 16

CHUNK = 128
N_PAD = 10112
ROWS_PER_TILE = N_PAD // NS
EDGES_PER_TILE = 10112
E_PAD = EDGES_PER_TILE * NC * NS
CHUNKS_PER_TILE = EDGES_PER_TILE // CHUNK
DEGW = 128


def _deg_body(col_hbm, zeros_hbm, ones_hbm, out_hbm, shared, idx_v, ones_v, sem):
    c = lax.axis_index("c")
    s = lax.axis_index("s")
    wid = s * NC + c
    r0 = s * ROWS_PER_TILE
    pltpu.sync_copy(zeros_hbm.at[pl.ds(r0, ROWS_PER_TILE), :],
                    shared.at[pl.ds(r0, ROWS_PER_TILE), :])
    pltpu.sync_copy(ones_hbm, ones_v)
    plsc.subcore_barrier()
    base = wid * EDGES_PER_TILE

    def body(j, carry):
        pltpu.sync_copy(col_hbm.at[pl.ds(base + j * CHUNK, CHUNK)], idx_v)
        pltpu.sync_copy(ones_v, shared.at[idx_v], add=True)
        return carry

    lax.fori_loop(0, CHUNKS_PER_TILE, body, 0)
    plsc.subcore_barrier()
    pltpu.sync_copy(shared.at[pl.ds(r0, ROWS_PER_TILE), :],
                    out_hbm.at[pl.ds(c * N_PAD + r0, ROWS_PER_TILE), :])


@functools.cache
def _deg_kernel():
    mesh = plsc.VectorSubcoreMesh(core_axis_name="c", subcore_axis_name="s")
    return pl.kernel(
        _deg_body,
        out_type=jax.ShapeDtypeStruct((NC * N_PAD, DEGW), jnp.float32),
        mesh=mesh,
        scratch_types=[
            pltpu.VMEM_SHARED((N_PAD, DEGW), jnp.float32),
            pltpu.VMEM((CHUNK,), jnp.int32),
            pltpu.VMEM((CHUNK, DEGW), jnp.float32),
            pltpu.SemaphoreType.DMA,
        ],
    )


def _agg_body(hs_hbm, row_hbm, col_hbm, zeros_hbm, out_hbm,
              shared, ridx_v, cidx_v, rows_v, sem):
    c = lax.axis_index("c")
    s = lax.axis_index("s")
    wid = s * NC + c
    r0 = s * ROWS_PER_TILE
    pltpu.sync_copy(zeros_hbm.at[pl.ds(r0, ROWS_PER_TILE), :],
                    shared.at[pl.ds(r0, ROWS_PER_TILE), :])
    plsc.subcore_barrier()
    base = wid * EDGES_PER_TILE

    def body(j, carry):
        e0 = base + j * CHUNK
        pltpu.sync_copy(row_hbm.at[pl.ds(e0, CHUNK)], ridx_v)
        pltpu.sync_copy(col_hbm.at[pl.ds(e0, CHUNK)], cidx_v)
        pltpu.async_copy(hs_hbm.at[ridx_v], rows_v, sem).wait()
        pltpu.sync_copy(rows_v, shared.at[cidx_v], add=True)
        return carry

    lax.fori_loop(0, CHUNKS_PER_TILE, body, 0)
    plsc.subcore_barrier()
    pltpu.sync_copy(shared.at[pl.ds(r0, ROWS_PER_TILE), :],
                    out_hbm.at[pl.ds(c * N_PAD + r0, ROWS_PER_TILE), :])


@functools.cache
def _agg_kernel():
    mesh = plsc.VectorSubcoreMesh(core_axis_name="c", subcore_axis_name="s")
    return pl.kernel(
        _agg_body,
        out_type=jax.ShapeDtypeStruct((NC * N_PAD, D), jnp.float32),
        mesh=mesh,
        scratch_types=[
            pltpu.VMEM_SHARED((N_PAD, D), jnp.float32),
            pltpu.VMEM((CHUNK,), jnp.int32),
            pltpu.VMEM((CHUNK,), jnp.int32),
            pltpu.VMEM((CHUNK, D), jnp.float32),
            pltpu.SemaphoreType.DMA,
        ],
    )


def _proj_body(x_ref, wp_ref, bp_ref, degp_ref, h_ref, hs_ref, dis_ref):
    deg = degp_ref[0:N_PAD, 0:1] + degp_ref[N_PAD:2 * N_PAD, 0:1]
    dis = lax.rsqrt(jnp.maximum(deg, 1.0))
    dis_ref[...] = dis
    h = jnp.dot(x_ref[...], wp_ref[...], preferred_element_type=jnp.float32)
    h = h + bp_ref[...]
    h_ref[...] = h
    hs_ref[...] = dis[:N] * h


_proj_kernel = pl.pallas_call(
    _proj_body,
    out_shape=(
        jax.ShapeDtypeStruct((N, D), jnp.float32),
        jax.ShapeDtypeStruct((N, D), jnp.float32),
        jax.ShapeDtypeStruct((N_PAD, 1), jnp.float32),
    ),
)


def _layer_body(aggp_ref, dis_ref, h_ref, w_ref, b_ref, hn_ref, hsn_ref):
    agg = aggp_ref[0:N, :] + aggp_ref[N_PAD:N_PAD + N, :]
    agg = agg * dis_ref[0:N, :]
    out = jnp.dot(agg, w_ref[...], preferred_element_type=jnp.float32)
    out = jnp.maximum(out + b_ref[...], 0.0) + h_ref[...]
    hn_ref[...] = out
    hsn_ref[...] = dis_ref[0:N, :] * out


_layer_kernel = pl.pallas_call(
    _layer_body,
    out_shape=(
        jax.ShapeDtypeStruct((N, D), jnp.float32),
        jax.ShapeDtypeStruct((N, D), jnp.float32),
    ),
)


def _pool_body(h_ref, batch_ref, wr1_ref, br1_ref, wr2_ref, br2_ref, out_ref):
    gids = lax.broadcasted_iota(jnp.int32, (G, N), 0)
    onehot_t = jnp.where(gids == batch_ref[...], 1.0, 0.0)
    sums = jnp.dot(onehot_t, h_ref[...], preferred_element_type=jnp.float32)
    counts = jnp.dot(onehot_t, jnp.ones((N, 1), jnp.float32),
                     preferred_element_type=jnp.float32)
    emb = sums / jnp.maximum(counts, 1.0)
    hid = jnp.dot(emb, wr1_ref[...], preferred_element_type=jnp.float32)
    hid = jnp.maximum(hid + br1_ref[...], 0.0)
    out = jnp.dot(hid, wr2_ref[...], preferred_element_type=jnp.float32)
    out_ref[...] = out + br2_ref[...]


_pool_kernel = pl.pallas_call(
    _pool_body,
    out_shape=jax.ShapeDtypeStruct((G, OUT), jnp.float32),
)


def kernel(x, edge_index, edge_attr, batch, Wp, bp, Wls, bls, Wr1, br1, Wr2, br2):
    del edge_attr
    row = edge_index[0]
    col = edge_index[1]
    pad = E_PAD - E
    row_p = jnp.concatenate([row, jnp.zeros((pad,), jnp.int32)])
    col_p = jnp.concatenate([col, jnp.full((pad,), N, jnp.int32)])

    zeros_n8 = jnp.zeros((N_PAD, DEGW), jnp.float32)
    zeros_nd = jnp.zeros((N_PAD, D), jnp.float32)
    ones_k8 = jnp.ones((CHUNK, DEGW), jnp.float32)

    degp = _deg_kernel()(col_p, zeros_n8, ones_k8)
    h, hs, dis = _proj_kernel(x, Wp, bp[None, :], degp)
    for i in range(L):
        aggp = _agg_kernel()(hs, row_p, col_p, zeros_nd)
        h, hs = _layer_kernel(aggp, dis, h, Wls[i], bls[i][None, :])
    return _pool_kernel(h, batch[None, :], Wr1, br1[None, :], Wr2, br2[None, :])
